# Initial kernel scaffold; baseline (speedup 1.0000x reference)
import jax
import jax.numpy as jnp
from jax.experimental import pallas as pl


def kernel(x, emb_table, W1, b1, W2, b2):
    raise NotImplementedError("write your pallas kernel here")


# speedup vs baseline: 15.9938x; 15.9938x over previous
import functools

import jax
import jax.numpy as jnp
from jax import lax
from jax.experimental import pallas as pl
from jax.experimental.pallas import tpu as pltpu
from jax.experimental.pallas import tpu_sc as plsc

B = 16384
S = 200
E = 32
NC = 2
NS = 16
NW = NC * NS
BPW = B // NW

CB = 8
JROWS = 2 * CB
NCHUNK = BPW // CB
HALF = S // 2


def _sc_pool_body(x_hbm, tbl_hbm, out_hbm, ibuf, gbuf, pooled_v, sem_i, sem_g):
    cid = lax.axis_index("c")
    sid = lax.axis_index("s")
    wid = sid * NC + cid
    row0 = wid * (BPW * 2)
    brow0 = wid * BPW

    def idx_start(c, buf):
        pltpu.async_copy(
            x_hbm.at[pl.ds(row0 + c * JROWS, JROWS)], ibuf.at[buf], sem_i)

    def idx_wait():
        pltpu.make_async_copy(
            x_hbm.at[pl.ds(row0, JROWS)], ibuf.at[0], sem_i).wait()

    def gather_start(buf):
        for j in range(JROWS):
            pltpu.async_copy(
                tbl_hbm.at[ibuf.at[buf, j]], gbuf.at[buf, j], sem_g)

    def gather_wait(buf):
        for j in range(JROWS):
            pltpu.make_async_copy(
                tbl_hbm.at[ibuf.at[buf, j]], gbuf.at[buf, j], sem_g).wait()

    idx_start(0, 0)
    idx_wait()
    gather_start(0)
    idx_start(1, 1)

    zero = jnp.zeros((16,), jnp.float32)

    def chunk_body(c, carry):
        buf = lax.rem(c, 2)
        gather_wait(buf)

        @pl.when(c + 2 < NCHUNK)
        def _():
            idx_start(c + 2, buf)

        @pl.when(c + 1 < NCHUNK)
        def _():
            idx_wait()
            gather_start(1 - buf)

        for b in range(CB):
            def t_body(t4, acc, b=b):
                a0, a1 = acc
                for k in range(4):
                    t = t4 * 4 + k
                    a0 = (a0
                          + gbuf[buf, 2 * b, t, pl.ds(0, 16)]
                          + gbuf[buf, 2 * b + 1, t, pl.ds(0, 16)])
                    a1 = (a1
                          + gbuf[buf, 2 * b, t, pl.ds(16, 16)]
                          + gbuf[buf, 2 * b + 1, t, pl.ds(16, 16)])
                return a0, a1

            a0, a1 = lax.fori_loop(0, HALF // 4, t_body, (zero, zero))
            r = c * CB + b
            pooled_v[r, pl.ds(0, 16)] = a0
            pooled_v[r, pl.ds(16, 16)] = a1
        return carry

    lax.fori_loop(0, NCHUNK, chunk_body, 0)
    pltpu.sync_copy(pooled_v, out_hbm.at[pl.ds(brow0, BPW)])


@jax.jit
def _sc_pool(x2d, emb_table):
    return pl.kernel(
        _sc_pool_body,
        out_type=jax.ShapeDtypeStruct((B, E), jnp.float32),
        mesh=plsc.VectorSubcoreMesh(
            core_axis_name="c", subcore_axis_name="s",
            num_cores=NC, num_subcores=NS),
        scratch_types=[
            pltpu.VMEM((2, JROWS, HALF), jnp.int32),
            pltpu.VMEM((2, JROWS, HALF, E), jnp.float32),
            pltpu.VMEM((BPW, E), jnp.float32),
            pltpu.SemaphoreType.DMA,
            pltpu.SemaphoreType.DMA,
        ],
        compiler_params=pltpu.CompilerParams(use_tc_tiling_on_sc=False),
    )(x2d, emb_table)


def _mlp_body(p_ref, w1_ref, b1_ref, w2_ref, b2_ref, o_ref):
    p = p_ref[...] * (1.0 / S)
    h = jnp.dot(p, w1_ref[...], preferred_element_type=jnp.float32)
    h = jnp.maximum(h + b1_ref[...], 0.0)
    z = jnp.sum(h * w2_ref[...], axis=1, keepdims=True) + b2_ref[...]
    o_ref[...] = jax.nn.sigmoid(z)


@jax.jit
def _tc_mlp(pooled_sum, W1, b1, W2, b2):
    return pl.pallas_call(
        _mlp_body,
        out_shape=jax.ShapeDtypeStruct((B, 1), jnp.float32),
    )(pooled_sum, W1, b1.reshape(1, E), W2.reshape(1, E), b2.reshape(1, 1))


def kernel(x, emb_table, W1, b1, W2, b2):
    x2d = x.astype(jnp.int32).reshape(2 * B, HALF)
    pooled_sum = _sc_pool(x2d, emb_table)
    return _tc_mlp(pooled_sum, W1, b1, W2, b2)
